# baseline (speedup 1.0000x reference)
import functools

import jax
import jax.numpy as jnp
from jax import lax
from jax.experimental import pallas as pl
from jax.experimental.pallas import tpu as pltpu
from jax.experimental.pallas import tpu_sc as plsc

N = 10000
E = 160000
D = 256
DH = 128
NC = 2
NS = 16
CH = 64
KB = 4
NB = 4
BC = 40
NCH = NB * BC
EPAD = NS * NCH * CH
ZR = CH
ZFULL = N // ZR
ZTAIL = N - ZFULL * ZR
AGG_ROWS = N + 8


def _sc_body(xlo, xhi, src_hbm, dst_hbm, out_lo, out_hi,
             sidx, didx, rows, agg_sh, *sems):
    c = lax.axis_index("c")
    s = lax.axis_index("s")

    def run_half(x_ref, out_ref):
        def my_chunks(fn, tail_fn):
            for r in range((ZFULL + NS) // NS):
                k = r * NS + s

                @pl.when(k < ZFULL)
                def _():
                    fn(pl.multiple_of(k * ZR, 8))

            @pl.when(s == ZFULL % NS)
            def _():
                tail_fn(ZFULL * ZR)

        def zfill(i, carry):
            rows[0, i // 8, pl.ds((i % 8) * 16, 16)] = jnp.zeros((16,), jnp.float32)
            return carry
        lax.fori_loop(0, CH * 8, zfill, None)
        my_chunks(
            lambda r0: pltpu.sync_copy(rows.at[0], agg_sh.at[pl.ds(r0, ZR)]),
            lambda r0: pltpu.sync_copy(rows.at[0, pl.ds(0, ZTAIL)],
                                       agg_sh.at[pl.ds(r0, ZTAIL)]))
        plsc.subcore_barrier()

        sem_g = sems[:KB]
        sem_s = sems[KB:]

        def scat_start(j, k, sem):
            pltpu.async_copy(rows.at[k], agg_sh.at[didx.at[j]], sem, add=True)

        def scat_wait(k, sem):
            pltpu.make_async_copy(rows.at[k], agg_sh.at[pl.ds(0, CH)],
                                  sem).wait()

        def stanza(jj, p, first):
            b0, b1 = 2 * p, 2 * p + 1
            if not first:
                scat_wait(b0, sem_s[b0])
            cpa = pltpu.async_copy(x_ref.at[sidx.at[jj]],
                                   rows.at[b0], sem_g[b0])
            if not first:
                scat_wait(b1, sem_s[b1])
            cpb = pltpu.async_copy(x_ref.at[sidx.at[jj + 1]],
                                   rows.at[b1], sem_g[b1])
            cpa.wait()
            scat_start(jj, b0, sem_s[b0])
            cpb.wait()
            scat_start(jj + 1, b1, sem_s[b1])

        def block(b, carry):
            pltpu.sync_copy(src_hbm.at[s * NB + b], sidx)
            pltpu.sync_copy(dst_hbm.at[s * NB + b], didx)
            for p in range(KB // 2):
                stanza(2 * p, p, True)

            def step(i, carry2):
                for p in range(KB // 2):
                    stanza(KB * i + 2 * p, p, False)
                return carry2
            lax.fori_loop(1, BC // KB, step, None)
            for k in range(KB):
                scat_wait(k, sem_s[k])
            return carry
        lax.fori_loop(0, NB, block, None)
        plsc.subcore_barrier()

        my_chunks(
            lambda r0: pltpu.sync_copy(agg_sh.at[pl.ds(r0, ZR)],
                                       out_ref.at[pl.ds(r0, ZR)]),
            lambda r0: pltpu.sync_copy(agg_sh.at[pl.ds(r0, ZTAIL)],
                                       out_ref.at[pl.ds(r0, ZTAIL)]))

    @pl.when(c == 0)
    def _():
        run_half(xlo, out_lo)

    @pl.when(c == 1)
    def _():
        run_half(xhi, out_hi)


@functools.cache
def _get_sc_agg():
    return pl.kernel(
        _sc_body,
        out_type=(jax.ShapeDtypeStruct((N, DH), jnp.float32),
                  jax.ShapeDtypeStruct((N, DH), jnp.float32)),
        mesh=plsc.VectorSubcoreMesh(core_axis_name="c", subcore_axis_name="s",
                                    num_cores=NC, num_subcores=NS),
        scratch_types=[
            pltpu.VMEM((BC, CH), jnp.int32),
            pltpu.VMEM((BC, CH), jnp.int32),
            pltpu.VMEM((KB, CH, DH), jnp.float32),
            pltpu.VMEM_SHARED((AGG_ROWS, DH), jnp.float32),
        ] + [pltpu.SemaphoreType.DMA] * (2 * KB),
    )


def _mlp_body(split_out, xl, xh, al, ah, w1, b1, g, be, w2, b2, *outs):
    h0l = xl[...] + al[...]
    h0h = xh[...] + ah[...]
    h1 = (jnp.dot(h0l, w1[:DH, :], preferred_element_type=jnp.float32)
          + jnp.dot(h0h, w1[DH:, :], preferred_element_type=jnp.float32)
          + b1[...])
    m = jnp.mean(h1, axis=0, keepdims=True)
    dcen = h1 - m
    v = jnp.mean(dcen * dcen, axis=0, keepdims=True)
    hn = g[...] * dcen * lax.rsqrt(v + 1e-5) + be[...]
    hr = jnp.maximum(hn, 0.0)
    y = jnp.maximum(
        jnp.dot(hr, w2[...], preferred_element_type=jnp.float32) + b2[...], 0.0)
    if split_out:
        outs[0][...] = y[:, :DH]
        outs[1][...] = y[:, DH:]
    else:
        outs[0][...] = y


_mlp_split = pl.pallas_call(
    functools.partial(_mlp_body, True),
    out_shape=(jax.ShapeDtypeStruct((N, DH), jnp.float32),
               jax.ShapeDtypeStruct((N, DH), jnp.float32)),
)

_mlp_full = pl.pallas_call(
    functools.partial(_mlp_body, False),
    out_shape=jax.ShapeDtypeStruct((N, D), jnp.float32),
)


def kernel(x, edge_index, W1_0, b1_0, g_0, be_0, W2_0, b2_0,
           W1_1, b1_1, g_1, be_1, W2_1, b2_1,
           W1_2, b1_2, g_2, be_2, W2_2, b2_2):
    src = edge_index[0]
    dst = edge_index[1]
    pad = EPAD - E
    src3 = jnp.concatenate(
        [src, jnp.zeros((pad,), jnp.int32)]).reshape(NS * NB, BC, CH)
    dst3 = jnp.concatenate(
        [dst, jnp.full((pad,), N, jnp.int32)]).reshape(NS * NB, BC, CH)

    xl = x[:, :DH]
    xh = x[:, DH:]
    params = [(W1_0, b1_0, g_0, be_0, W2_0, b2_0),
              (W1_1, b1_1, g_1, be_1, W2_1, b2_1),
              (W1_2, b1_2, g_2, be_2, W2_2, b2_2)]
    for i, (w1, b1, g, be, w2, b2) in enumerate(params):
        al, ah = _get_sc_agg()(xl, xh, src3, dst3)
        args = (xl, xh, al, ah, w1, b1.reshape(1, -1), g.reshape(1, -1),
                be.reshape(1, -1), w2, b2.reshape(1, -1))
        if i < 2:
            xl, xh = _mlp_split(*args)
        else:
            return _mlp_full(*args)


# speedup vs baseline: 1.0238x; 1.0238x over previous
import functools

import jax
import jax.numpy as jnp
from jax import lax
from jax.experimental import pallas as pl
from jax.experimental.pallas import tpu as pltpu
from jax.experimental.pallas import tpu_sc as plsc

N = 10000
E = 160000
D = 256
DH = 128
NC = 2
NS = 16
CH = 64
KB = 4
NB = 4
BC = 40
NCH = NB * BC
EPAD = NS * NCH * CH
ZR = CH
ZFULL = N // ZR
ZTAIL = N - ZFULL * ZR
AGG_ROWS = N + 8


def _sc_body(xlo, xhi, src_hbm, dst_hbm, out_lo, out_hi,
             sidx, didx, rows, agg_sh, *sems):
    c = lax.axis_index("c")
    s = lax.axis_index("s")

    def run_half(x_ref, out_ref):
        def my_chunks(fn, tail_fn):
            for r in range((ZFULL + NS) // NS):
                k = r * NS + s

                @pl.when(k < ZFULL)
                def _():
                    fn(pl.multiple_of(k * ZR, 8))

            @pl.when(s == ZFULL % NS)
            def _():
                tail_fn(ZFULL * ZR)

        def zfill(i, carry):
            rows[0, i // 8, pl.ds((i % 8) * 16, 16)] = jnp.zeros((16,), jnp.float32)
            return carry
        lax.fori_loop(0, CH * 8, zfill, None)
        my_chunks(
            lambda r0: pltpu.sync_copy(rows.at[0], agg_sh.at[pl.ds(r0, ZR)]),
            lambda r0: pltpu.sync_copy(rows.at[0, pl.ds(0, ZTAIL)],
                                       agg_sh.at[pl.ds(r0, ZTAIL)]))
        plsc.subcore_barrier()

        sem_g = sems[:KB]
        sem_s = sems[KB:]

        def scat_start(j, k, sem):
            pltpu.async_copy(rows.at[k], agg_sh.at[didx.at[j]], sem, add=True)

        def scat_wait(k, sem):
            pltpu.make_async_copy(rows.at[k], agg_sh.at[pl.ds(0, CH)],
                                  sem).wait()

        def stanza(jj, p, first):
            b0, b1 = 2 * p, 2 * p + 1
            if not first:
                scat_wait(b0, sem_s[b0])
                scat_wait(b1, sem_s[b1])
            cpa = pltpu.async_copy(x_ref.at[sidx.at[jj]],
                                   rows.at[b0], sem_g[b0])
            cpb = pltpu.async_copy(x_ref.at[sidx.at[jj + 1]],
                                   rows.at[b1], sem_g[b1])
            cpa.wait()
            scat_start(jj, b0, sem_s[b0])
            cpb.wait()
            scat_start(jj + 1, b1, sem_s[b1])

        def block(b, carry):
            pltpu.sync_copy(src_hbm.at[s * NB + b], sidx)
            pltpu.sync_copy(dst_hbm.at[s * NB + b], didx)
            for k in range(KB):
                scat_start(BC, k, sem_s[k])

            def step(i, carry2):
                for p in range(KB // 2):
                    stanza(KB * i + 2 * p, p, False)
                return carry2
            lax.fori_loop(0, BC // KB, step, None)
            for k in range(KB):
                scat_wait(k, sem_s[k])
            return carry
        lax.fori_loop(0, NB, block, None)
        plsc.subcore_barrier()

        my_chunks(
            lambda r0: pltpu.sync_copy(agg_sh.at[pl.ds(r0, ZR)],
                                       out_ref.at[pl.ds(r0, ZR)]),
            lambda r0: pltpu.sync_copy(agg_sh.at[pl.ds(r0, ZTAIL)],
                                       out_ref.at[pl.ds(r0, ZTAIL)]))

    @pl.when(c == 0)
    def _():
        run_half(xlo, out_lo)

    @pl.when(c == 1)
    def _():
        run_half(xhi, out_hi)


@functools.cache
def _get_sc_agg():
    return pl.kernel(
        _sc_body,
        out_type=(jax.ShapeDtypeStruct((N, DH), jnp.float32),
                  jax.ShapeDtypeStruct((N, DH), jnp.float32)),
        mesh=plsc.VectorSubcoreMesh(core_axis_name="c", subcore_axis_name="s",
                                    num_cores=NC, num_subcores=NS),
        scratch_types=[
            pltpu.VMEM((BC, CH), jnp.int32),
            pltpu.VMEM((BC + 1, CH), jnp.int32),
            pltpu.VMEM((KB, CH, DH), jnp.float32),
            pltpu.VMEM_SHARED((AGG_ROWS, DH), jnp.float32),
        ] + [pltpu.SemaphoreType.DMA] * (2 * KB),
    )


def _mlp_body(split_out, xl, xh, al, ah, w1, b1, g, be, w2, b2, *outs):
    h0l = xl[...] + al[...]
    h0h = xh[...] + ah[...]
    h1 = (jnp.dot(h0l, w1[:DH, :], preferred_element_type=jnp.float32)
          + jnp.dot(h0h, w1[DH:, :], preferred_element_type=jnp.float32)
          + b1[...])
    m = jnp.mean(h1, axis=0, keepdims=True)
    dcen = h1 - m
    v = jnp.mean(dcen * dcen, axis=0, keepdims=True)
    hn = g[...] * dcen * lax.rsqrt(v + 1e-5) + be[...]
    hr = jnp.maximum(hn, 0.0)
    y = jnp.maximum(
        jnp.dot(hr, w2[...], preferred_element_type=jnp.float32) + b2[...], 0.0)
    if split_out:
        outs[0][...] = y[:, :DH]
        outs[1][...] = y[:, DH:]
    else:
        outs[0][...] = y


_mlp_split = pl.pallas_call(
    functools.partial(_mlp_body, True),
    out_shape=(jax.ShapeDtypeStruct((N, DH), jnp.float32),
               jax.ShapeDtypeStruct((N, DH), jnp.float32)),
)

_mlp_full = pl.pallas_call(
    functools.partial(_mlp_body, False),
    out_shape=jax.ShapeDtypeStruct((N, D), jnp.float32),
)


def kernel(x, edge_index, W1_0, b1_0, g_0, be_0, W2_0, b2_0,
           W1_1, b1_1, g_1, be_1, W2_1, b2_1,
           W1_2, b1_2, g_2, be_2, W2_2, b2_2):
    src = edge_index[0]
    dst = edge_index[1]
    pad = EPAD - E
    src3 = jnp.concatenate(
        [src, jnp.zeros((pad,), jnp.int32)]).reshape(NS * NB, BC, CH)
    dst3 = jnp.concatenate(
        [dst, jnp.full((pad,), N, jnp.int32)]).reshape(NS * NB, BC, CH)
    dst3 = jnp.concatenate(
        [dst3, jnp.full((NS * NB, 1, CH), N, jnp.int32)], axis=1)

    xl = x[:, :DH]
    xh = x[:, DH:]
    params = [(W1_0, b1_0, g_0, be_0, W2_0, b2_0),
              (W1_1, b1_1, g_1, be_1, W2_1, b2_1),
              (W1_2, b1_2, g_2, be_2, W2_2, b2_2)]
    for i, (w1, b1, g, be, w2, b2) in enumerate(params):
        al, ah = _get_sc_agg()(xl, xh, src3, dst3)
        args = (xl, xh, al, ah, w1, b1.reshape(1, -1), g.reshape(1, -1),
                be.reshape(1, -1), w2, b2.reshape(1, -1))
        if i < 2:
            xl, xh = _mlp_split(*args)
        else:
            return _mlp_full(*args)


# speedup vs baseline: 1.0241x; 1.0003x over previous
import functools

import jax
import jax.numpy as jnp
from jax import lax
from jax.experimental import pallas as pl
from jax.experimental.pallas import tpu as pltpu
from jax.experimental.pallas import tpu_sc as plsc

N = 10000
E = 160000
D = 256
DH = 128
NC = 2
NS = 16
CH = 64
KB = 4
NB = 4
BC = 40
NCH = NB * BC
EPAD = NS * NCH * CH
ZR = CH
ZFULL = N // ZR
ZTAIL = N - ZFULL * ZR
AGG_ROWS = N + 8


def _sc_body(xlo, xhi, src_hbm, dst_hbm, out_lo, out_hi,
             sidx, didx, rows, agg_sh, *sems):
    c = lax.axis_index("c")
    s = lax.axis_index("s")

    def run_half(x_ref, out_ref):
        def my_chunks(fn, tail_fn):
            for r in range((ZFULL + NS) // NS):
                k = r * NS + s

                @pl.when(k < ZFULL)
                def _():
                    fn(pl.multiple_of(k * ZR, 8))

            @pl.when(s == ZFULL % NS)
            def _():
                tail_fn(ZFULL * ZR)

        def zfill(i, carry):
            rows[0, i // 8, pl.ds((i % 8) * 16, 16)] = jnp.zeros((16,), jnp.float32)
            return carry
        lax.fori_loop(0, CH * 8, zfill, None)
        my_chunks(
            lambda r0: pltpu.sync_copy(rows.at[0], agg_sh.at[pl.ds(r0, ZR)]),
            lambda r0: pltpu.sync_copy(rows.at[0, pl.ds(0, ZTAIL)],
                                       agg_sh.at[pl.ds(r0, ZTAIL)]))
        plsc.subcore_barrier()

        sem_g = sems[:KB]
        sem_s = sems[KB:]

        def scat_start(j, k, sem):
            pltpu.async_copy(rows.at[k], agg_sh.at[didx.at[j]], sem, add=True)

        def scat_wait(k, sem):
            pltpu.make_async_copy(rows.at[k], agg_sh.at[pl.ds(0, CH)],
                                  sem).wait()

        def stanza(jj, p, first):
            b0, b1 = 2 * p, 2 * p + 1
            if not first:
                scat_wait(b0, sem_s[b0])
                scat_wait(b1, sem_s[b1])
            cpa = pltpu.async_copy(x_ref.at[sidx.at[jj]],
                                   rows.at[b0], sem_g[b0])
            cpb = pltpu.async_copy(x_ref.at[sidx.at[jj + 1]],
                                   rows.at[b1], sem_g[b1])
            cpa.wait()
            scat_start(jj, b0, sem_s[b0])
            cpb.wait()
            scat_start(jj + 1, b1, sem_s[b1])

        def block(b, carry):
            pltpu.sync_copy(src_hbm.at[s * NB + b], sidx)
            pltpu.sync_copy(dst_hbm.at[s * NB + b], didx)
            for k in range(KB):
                scat_start(BC, k, sem_s[k])

            def step(i, carry2):
                for p in range(KB // 2):
                    stanza(KB * i + 2 * p, p, False)
                return carry2
            lax.fori_loop(0, BC // KB, step, None)
            for k in range(KB):
                scat_wait(k, sem_s[k])
            return carry
        lax.fori_loop(0, NB, block, None)
        plsc.subcore_barrier()

        my_chunks(
            lambda r0: pltpu.sync_copy(agg_sh.at[pl.ds(r0, ZR)],
                                       out_ref.at[pl.ds(r0, ZR)]),
            lambda r0: pltpu.sync_copy(agg_sh.at[pl.ds(r0, ZTAIL)],
                                       out_ref.at[pl.ds(r0, ZTAIL)]))

    @pl.when(c == 0)
    def _():
        run_half(xlo, out_lo)

    @pl.when(c == 1)
    def _():
        run_half(xhi, out_hi)


@functools.cache
def _get_sc_agg():
    return pl.kernel(
        _sc_body,
        out_type=(jax.ShapeDtypeStruct((N, DH), jnp.float32),
                  jax.ShapeDtypeStruct((N, DH), jnp.float32)),
        mesh=plsc.VectorSubcoreMesh(core_axis_name="c", subcore_axis_name="s",
                                    num_cores=NC, num_subcores=NS),
        scratch_types=[
            pltpu.VMEM((BC, CH), jnp.int32),
            pltpu.VMEM((BC + 1, CH), jnp.int32),
            pltpu.VMEM((KB, CH, DH), jnp.float32),
            pltpu.VMEM_SHARED((AGG_ROWS, DH), jnp.float32),
        ] + [pltpu.SemaphoreType.DMA] * (2 * KB),
    )


RB = 1000
NBLK = N // RB


def _mlp_body(split_out, xl, xh, al, ah, w1, b1, g, be, w2, b2, *rest):
    outs = rest[:-2]
    h1_s, acc = rest[-2:]
    ph = pl.program_id(0)
    i = pl.program_id(1)

    @pl.when(ph == 0)
    def _():
        h0l = xl[...] + al[...]
        h0h = xh[...] + ah[...]
        h1 = (jnp.dot(h0l, w1[:DH, :], preferred_element_type=jnp.float32)
              + jnp.dot(h0h, w1[DH:, :], preferred_element_type=jnp.float32)
              + b1[...])
        h1_s[pl.ds(i * RB, RB), :] = h1

        @pl.when(i == 0)
        def _():
            acc[...] = jnp.zeros_like(acc)
        acc[0:1, :] += jnp.sum(h1, axis=0, keepdims=True)
        acc[1:2, :] += jnp.sum(h1 * h1, axis=0, keepdims=True)

    @pl.when(ph == 1)
    def _():
        m = acc[0:1, :] * (1.0 / N)
        v = acc[1:2, :] * (1.0 / N) - m * m
        h1 = h1_s[pl.ds(i * RB, RB), :]
        hn = g[...] * (h1 - m) * lax.rsqrt(v + 1e-5) + be[...]
        hr = jnp.maximum(hn, 0.0)
        y = jnp.maximum(
            jnp.dot(hr, w2[...], preferred_element_type=jnp.float32)
            + b2[...], 0.0)
        if split_out:
            outs[0][...] = y[:, :DH]
            outs[1][...] = y[:, DH:]
        else:
            outs[0][...] = y


def _in_phase0(ph, i):
    return (jnp.where(ph == 0, i, 0), 0)


def _out_phase1(ph, i):
    return (jnp.where(ph == 1, i, 0), 0)


def _const(ph, i):
    return (0, 0)


_MLP_IN_SPECS = [
    pl.BlockSpec((RB, DH), _in_phase0),
    pl.BlockSpec((RB, DH), _in_phase0),
    pl.BlockSpec((RB, DH), _in_phase0),
    pl.BlockSpec((RB, DH), _in_phase0),
    pl.BlockSpec((D, D), _const),
    pl.BlockSpec((1, D), _const),
    pl.BlockSpec((1, D), _const),
    pl.BlockSpec((1, D), _const),
    pl.BlockSpec((D, D), _const),
    pl.BlockSpec((1, D), _const),
]

_MLP_SCRATCH = [
    pltpu.VMEM((N, D), jnp.float32),
    pltpu.VMEM((2, D), jnp.float32),
]

_mlp_split = pl.pallas_call(
    functools.partial(_mlp_body, True),
    grid=(2, NBLK),
    in_specs=_MLP_IN_SPECS,
    out_specs=(pl.BlockSpec((RB, DH), _out_phase1),
               pl.BlockSpec((RB, DH), _out_phase1)),
    out_shape=(jax.ShapeDtypeStruct((N, DH), jnp.float32),
               jax.ShapeDtypeStruct((N, DH), jnp.float32)),
    scratch_shapes=_MLP_SCRATCH,
)

_mlp_full = pl.pallas_call(
    functools.partial(_mlp_body, False),
    grid=(2, NBLK),
    in_specs=_MLP_IN_SPECS,
    out_specs=pl.BlockSpec((RB, D), _out_phase1),
    out_shape=jax.ShapeDtypeStruct((N, D), jnp.float32),
    scratch_shapes=_MLP_SCRATCH,
)


def kernel(x, edge_index, W1_0, b1_0, g_0, be_0, W2_0, b2_0,
           W1_1, b1_1, g_1, be_1, W2_1, b2_1,
           W1_2, b1_2, g_2, be_2, W2_2, b2_2):
    src = edge_index[0]
    dst = edge_index[1]
    pad = EPAD - E
    src3 = jnp.concatenate(
        [src, jnp.zeros((pad,), jnp.int32)]).reshape(NS * NB, BC, CH)
    dst3 = jnp.concatenate(
        [dst, jnp.full((pad,), N, jnp.int32)]).reshape(NS * NB, BC, CH)
    dst3 = jnp.concatenate(
        [dst3, jnp.full((NS * NB, 1, CH), N, jnp.int32)], axis=1)

    xl = x[:, :DH]
    xh = x[:, DH:]
    params = [(W1_0, b1_0, g_0, be_0, W2_0, b2_0),
              (W1_1, b1_1, g_1, be_1, W2_1, b2_1),
              (W1_2, b1_2, g_2, be_2, W2_2, b2_2)]
    for i, (w1, b1, g, be, w2, b2) in enumerate(params):
        al, ah = _get_sc_agg()(xl, xh, src3, dst3)
        args = (xl, xh, al, ah, w1, b1.reshape(1, -1), g.reshape(1, -1),
                be.reshape(1, -1), w2, b2.reshape(1, -1))
        if i < 2:
            xl, xh = _mlp_split(*args)
        else:
            return _mlp_full(*args)


# speedup vs baseline: 1.0316x; 1.0073x over previous
import functools

import jax
import jax.numpy as jnp
from jax import lax
from jax.experimental import pallas as pl
from jax.experimental.pallas import tpu as pltpu
from jax.experimental.pallas import tpu_sc as plsc

N = 10000
E = 160000
D = 256
DH = 128
NC = 2
NS = 16
CH = 64
KB = 4
NB = 4
BC = 40
NCH = NB * BC
EPAD = NS * NCH * CH
ZR = CH
ZFULL = N // ZR
ZTAIL = N - ZFULL * ZR
AGG_ROWS = N + 8


def _sc_body(xlo, xhi, src_hbm, dst_hbm, out_lo, out_hi,
             sidx, didx, rows, agg_sh, *sems):
    c = lax.axis_index("c")
    s = lax.axis_index("s")

    def run_half(x_ref, out_ref):
        def my_chunks(fn, tail_fn):
            for r in range((ZFULL + NS) // NS):
                k = r * NS + s

                @pl.when(k < ZFULL)
                def _():
                    fn(pl.multiple_of(k * ZR, 8))

            @pl.when(s == ZFULL % NS)
            def _():
                tail_fn(ZFULL * ZR)

        def zfill(i, carry):
            rows[0, i // 8, pl.ds((i % 8) * 16, 16)] = jnp.zeros((16,), jnp.float32)
            return carry
        lax.fori_loop(0, CH * 8, zfill, None)
        my_chunks(
            lambda r0: pltpu.sync_copy(rows.at[0], agg_sh.at[pl.ds(r0, ZR)]),
            lambda r0: pltpu.sync_copy(rows.at[0, pl.ds(0, ZTAIL)],
                                       agg_sh.at[pl.ds(r0, ZTAIL)]))
        plsc.subcore_barrier()

        sem_g = sems[:KB]
        sem_s = sems[KB:]

        def scat_start(j, k, sem):
            pltpu.async_copy(rows.at[k], agg_sh.at[didx.at[j]], sem, add=True)

        def scat_wait(k, sem):
            pltpu.make_async_copy(rows.at[k], agg_sh.at[pl.ds(0, CH)],
                                  sem).wait()

        def stanza(jj, p):
            b0, b1 = 2 * p, 2 * p + 1
            scat_wait(b0, sem_s[b0])
            scat_wait(b1, sem_s[b1])
            cpa = pltpu.async_copy(x_ref.at[sidx.at[jj]],
                                   rows.at[b0], sem_g[b0])
            cpb = pltpu.async_copy(x_ref.at[sidx.at[jj + 1]],
                                   rows.at[b1], sem_g[b1])
            cpa.wait()
            scat_start(jj, b0, sem_s[b0])
            cpb.wait()
            scat_start(jj + 1, b1, sem_s[b1])

        def block(b, carry):
            pltpu.sync_copy(src_hbm.at[s * NB + b], sidx)
            pltpu.sync_copy(dst_hbm.at[s * NB + b], didx)
            for k in range(KB):
                scat_start(BC, k, sem_s[k])

            def step(i, carry2):
                for p in range(KB // 2):
                    stanza(KB * i + 2 * p, p)
                return carry2
            lax.fori_loop(0, BC // KB, step, None)
            for k in range(KB):
                scat_wait(k, sem_s[k])
            return carry
        lax.fori_loop(0, NB, block, None)
        plsc.subcore_barrier()

        my_chunks(
            lambda r0: pltpu.sync_copy(agg_sh.at[pl.ds(r0, ZR)],
                                       out_ref.at[pl.ds(r0, ZR)]),
            lambda r0: pltpu.sync_copy(agg_sh.at[pl.ds(r0, ZTAIL)],
                                       out_ref.at[pl.ds(r0, ZTAIL)]))

    @pl.when(c == 0)
    def _():
        run_half(xlo, out_lo)

    @pl.when(c == 1)
    def _():
        run_half(xhi, out_hi)


@functools.cache
def _get_sc_agg():
    return pl.kernel(
        _sc_body,
        out_type=(jax.ShapeDtypeStruct((N, DH), jnp.float32),
                  jax.ShapeDtypeStruct((N, DH), jnp.float32)),
        mesh=plsc.VectorSubcoreMesh(core_axis_name="c", subcore_axis_name="s",
                                    num_cores=NC, num_subcores=NS),
        scratch_types=[
            pltpu.VMEM((BC, CH), jnp.int32),
            pltpu.VMEM((BC + 1, CH), jnp.int32),
            pltpu.VMEM((KB, CH, DH), jnp.float32),
            pltpu.VMEM_SHARED((AGG_ROWS, DH), jnp.float32),
        ] + [pltpu.SemaphoreType.DMA] * (2 * KB),
    )


RB = 1000
NBLK = N // RB


def _mlp_body(split_out, xl, xh, al, ah, w1, b1, g, be, w2, b2, *rest):
    outs = rest[:-2]
    h1_s, acc = rest[-2:]
    ph = pl.program_id(0)
    i = pl.program_id(1)

    @pl.when(ph == 0)
    def _():
        h0l = xl[...] + al[...]
        h0h = xh[...] + ah[...]
        h1 = (jnp.dot(h0l, w1[:DH, :], preferred_element_type=jnp.float32)
              + jnp.dot(h0h, w1[DH:, :], preferred_element_type=jnp.float32)
              + b1[...])
        h1_s[pl.ds(i * RB, RB), :] = h1

        @pl.when(i == 0)
        def _():
            acc[...] = jnp.zeros_like(acc)
        acc[0:1, :] += jnp.sum(h1, axis=0, keepdims=True)
        acc[1:2, :] += jnp.sum(h1 * h1, axis=0, keepdims=True)

    @pl.when(ph == 1)
    def _():
        m = acc[0:1, :] * (1.0 / N)
        v = acc[1:2, :] * (1.0 / N) - m * m
        h1 = h1_s[pl.ds(i * RB, RB), :]
        hn = g[...] * (h1 - m) * lax.rsqrt(v + 1e-5) + be[...]
        hr = jnp.maximum(hn, 0.0)
        y = jnp.maximum(
            jnp.dot(hr, w2[...], preferred_element_type=jnp.float32)
            + b2[...], 0.0)
        if split_out:
            outs[0][...] = y[:, :DH]
            outs[1][...] = y[:, DH:]
        else:
            outs[0][...] = y


def _in_phase0(ph, i):
    return (jnp.where(ph == 0, i, 0), 0)


def _out_phase1(ph, i):
    return (jnp.where(ph == 1, i, 0), 0)


def _const(ph, i):
    return (0, 0)


_MLP_IN_SPECS = [
    pl.BlockSpec((RB, DH), _in_phase0),
    pl.BlockSpec((RB, DH), _in_phase0),
    pl.BlockSpec((RB, DH), _in_phase0),
    pl.BlockSpec((RB, DH), _in_phase0),
    pl.BlockSpec((D, D), _const),
    pl.BlockSpec((1, D), _const),
    pl.BlockSpec((1, D), _const),
    pl.BlockSpec((1, D), _const),
    pl.BlockSpec((D, D), _const),
    pl.BlockSpec((1, D), _const),
]

_MLP_SCRATCH = [
    pltpu.VMEM((N, D), jnp.float32),
    pltpu.VMEM((2, D), jnp.float32),
]

_mlp_split = pl.pallas_call(
    functools.partial(_mlp_body, True),
    grid=(2, NBLK),
    in_specs=_MLP_IN_SPECS,
    out_specs=(pl.BlockSpec((RB, DH), _out_phase1),
               pl.BlockSpec((RB, DH), _out_phase1)),
    out_shape=(jax.ShapeDtypeStruct((N, DH), jnp.float32),
               jax.ShapeDtypeStruct((N, DH), jnp.float32)),
    scratch_shapes=_MLP_SCRATCH,
)

_mlp_full = pl.pallas_call(
    functools.partial(_mlp_body, False),
    grid=(2, NBLK),
    in_specs=_MLP_IN_SPECS,
    out_specs=pl.BlockSpec((RB, D), _out_phase1),
    out_shape=jax.ShapeDtypeStruct((N, D), jnp.float32),
    scratch_shapes=_MLP_SCRATCH,
)


def kernel(x, edge_index, W1_0, b1_0, g_0, be_0, W2_0, b2_0,
           W1_1, b1_1, g_1, be_1, W2_1, b2_1,
           W1_2, b1_2, g_2, be_2, W2_2, b2_2):
    src = edge_index[0]
    dst = edge_index[1]
    pad = EPAD - E
    src3 = jnp.concatenate(
        [src, jnp.zeros((pad,), jnp.int32)]).reshape(NS * NB, BC, CH)
    dst3 = jnp.concatenate(
        [dst, jnp.full((pad,), N, jnp.int32)]).reshape(NS * NB, BC, CH)
    dst3 = jnp.concatenate(
        [dst3, jnp.full((NS * NB, 1, CH), N, jnp.int32)], axis=1)

    xl = x[:, :DH]
    xh = x[:, DH:]
    params = [(W1_0, b1_0, g_0, be_0, W2_0, b2_0),
              (W1_1, b1_1, g_1, be_1, W2_1, b2_1),
              (W1_2, b1_2, g_2, be_2, W2_2, b2_2)]
    for i, (w1, b1, g, be, w2, b2) in enumerate(params):
        al, ah = _get_sc_agg()(xl, xh, src3, dst3)
        args = (xl, xh, al, ah, w1, b1.reshape(1, -1), g.reshape(1, -1),
                be.reshape(1, -1), w2, b2.reshape(1, -1))
        if i < 2:
            xl, xh = _mlp_split(*args)
        else:
            return _mlp_full(*args)
